# baseline (speedup 1.0000x reference)
import functools

import jax
import jax.numpy as jnp
from jax import lax
from jax.experimental import pallas as pl
from jax.experimental.pallas import tpu as pltpu
from jax.experimental.pallas import tpu_sc as plsc

_N = 10000
_E = 320000
_F_IN = 128
_H = 32
_C = 16

_NC = 2
_NS = 16
_NW = _NC * _NS
_EPW = _E // _NW
_CH = 128
_NCHUNK = _EPW // _CH
_TAIL = _EPW - _NCHUNK * _CH
_NBUF = 6
_DEPTH = 3
_NP = 10240
_RPS = _NP // _NS


def _make_sc_segsum(K: int):
    mesh = plsc.VectorSubcoreMesh(
        core_axis_name="c", subcore_axis_name="s",
        num_cores=_NC, num_subcores=_NS)

    def body(t_hbm, ei_hbm, ea_hbm, out_hbm,
             src_v, dst_v, ea_v, rows, acc, t_sp, gsem, ssem):
        cid = lax.axis_index("c")
        sid = lax.axis_index("s")
        wid = sid * _NC + cid

        def zrow(r, _):
            for f in range(0, K, 16):
                rows[0][r, pl.ds(f, 16)] = jnp.zeros((16,), jnp.float32)
            return 0

        lax.fori_loop(0, _CH, zrow, 0)
        for i in range(_RPS // _CH):
            pltpu.sync_copy(rows[0],
                            acc.at[pl.ds(sid * _RPS + i * _CH, _CH)])

        pltpu.sync_copy(t_hbm.at[pl.ds(sid * _RPS, _RPS)],
                        t_sp.at[pl.ds(sid * _RPS, _RPS)])

        base = wid * _EPW
        pltpu.sync_copy(ei_hbm.at[0, pl.ds(base, _EPW)], src_v)
        pltpu.sync_copy(ei_hbm.at[1, pl.ds(base, _EPW)], dst_v)
        pltpu.sync_copy(ea_hbm.at[pl.ds(base, _EPW)], ea_v)
        plsc.subcore_barrier()

        for b in range(_DEPTH):
            pltpu.async_copy(t_sp.at[src_v.at[pl.ds(b * _CH, _CH)]], rows[b], gsem[b])

        def step(c, b):
            pltpu.make_async_copy(
                t_sp.at[src_v.at[pl.ds(c * _CH, _CH)]], rows[b], gsem[b]).wait()
            for g in range(0, _CH, 16):
                ev = ea_v[pl.ds(c * _CH + g, 16)]
                for j in range(16):
                    r = g + j
                    s = ev[j]
                    for f in range(0, K, 16):
                        rows[b][r, pl.ds(f, 16)] = rows[b][r, pl.ds(f, 16)] * s
            pltpu.async_copy(rows[b], acc.at[dst_v.at[pl.ds(c * _CH, _CH)]], ssem[b],
                             add=True)
            cn = c + _DEPTH
            bn = (b + _DEPTH) % _NBUF
            co = c - (_NBUF - _DEPTH)

            @pl.when(cn < _NCHUNK)
            def _():
                @pl.when(co >= 0)
                def _():
                    pltpu.make_async_copy(
                        rows[bn], acc.at[dst_v.at[pl.ds(co * _CH, _CH)]], ssem[bn]).wait()
                pltpu.async_copy(t_sp.at[src_v.at[pl.ds(cn * _CH, _CH)]], rows[bn], gsem[bn])

        def macro(m, _):
            for b in range(_NBUF):
                step(m * _NBUF + b, b)
            return 0

        lax.fori_loop(0, _NCHUNK // _NBUF, macro, 0)

        for b in range(_NBUF):
            cc = _NCHUNK - _NBUF + b
            pltpu.make_async_copy(
                rows[b], acc.at[dst_v.at[pl.ds(cc * _CH, _CH)]], ssem[b]).wait()

        tb = _NCHUNK * _CH
        pltpu.async_copy(
            t_sp.at[src_v.at[pl.ds(tb, _TAIL)]],
            rows[0].at[pl.ds(0, _TAIL)], gsem[0]).wait()
        ev = ea_v[pl.ds(tb, 16)]
        for j in range(_TAIL):
            for f in range(0, K, 16):
                rows[0][j, pl.ds(f, 16)] = rows[0][j, pl.ds(f, 16)] * ev[j]
        pltpu.async_copy(
            rows[0].at[pl.ds(0, _TAIL)],
            acc.at[dst_v.at[pl.ds(tb, _TAIL)]], ssem[0], add=True).wait()

        plsc.subcore_barrier()

        pltpu.sync_copy(acc.at[pl.ds(sid * _RPS, _RPS)],
                        out_hbm.at[cid, pl.ds(sid * _RPS, _RPS)])

    return pl.kernel(
        body,
        out_type=jax.ShapeDtypeStruct((_NC, _NP, K), jnp.float32),
        mesh=mesh,
        compiler_params=pltpu.CompilerParams(use_tc_tiling_on_sc=False),
        scratch_types=[
            pltpu.VMEM((_EPW,), jnp.int32),
            pltpu.VMEM((_EPW,), jnp.int32),
            pltpu.VMEM((_EPW,), jnp.float32),
            [pltpu.VMEM((_CH, K), jnp.float32) for _ in range(_NBUF)],
            pltpu.VMEM_SHARED((_NP, K), jnp.float32),
            pltpu.VMEM_SHARED((_NP, K), jnp.float32),
            [pltpu.SemaphoreType.DMA for _ in range(_NBUF)],
            [pltpu.SemaphoreType.DMA for _ in range(_NBUF)],
        ],
        name=f"sc_segsum_k{K}",
    )


_sc_segsum_h = _make_sc_segsum(_H)
_sc_segsum_c = _make_sc_segsum(_C)


def _dense1_body(x_ref, wrel_ref, wroot_ref, xw_ref, xr_ref):
    x = x_ref[...]
    xw_ref[...] = jnp.dot(x, wrel_ref[...],
                          preferred_element_type=jnp.float32)
    xr_ref[...] = jnp.dot(x, wroot_ref[...],
                          preferred_element_type=jnp.float32)


def _dense2_body(parts_ref, xr_ref, b1_ref, wrel_ref, wroot_ref,
                 hw_ref, hr_ref):
    s = parts_ref[0] + parts_ref[1] + xr_ref[...] + b1_ref[...]
    h = jnp.maximum(s, 0.0)
    hw_ref[...] = jnp.dot(h, wrel_ref[...],
                          preferred_element_type=jnp.float32)
    hr_ref[...] = jnp.dot(h, wroot_ref[...],
                          preferred_element_type=jnp.float32)


def _out_body(parts_ref, hr_ref, b2_ref, o_ref):
    t = parts_ref[0] + parts_ref[1] + hr_ref[...] + b2_ref[...]
    m = jnp.max(t, axis=1, keepdims=True)
    lse = jnp.log(jnp.sum(jnp.exp(t - m), axis=1, keepdims=True)) + m
    o_ref[...] = t - lse


_BN = 1000


def kernel(x, edge_index, edge_attr, W1_rel, b1, W1_root, W2_rel, b2,
           W2_root):
    ei = edge_index.astype(jnp.int32)
    grid = _N // _BN

    xw1, xr1 = pl.pallas_call(
        _dense1_body,
        grid=(grid,),
        in_specs=[
            pl.BlockSpec((_BN, _F_IN), lambda i: (i, 0)),
            pl.BlockSpec((_F_IN, _H), lambda i: (0, 0)),
            pl.BlockSpec((_F_IN, _H), lambda i: (0, 0)),
        ],
        out_specs=[
            pl.BlockSpec((_BN, _H), lambda i: (i, 0)),
            pl.BlockSpec((_BN, _H), lambda i: (i, 0)),
        ],
        out_shape=[jax.ShapeDtypeStruct((_NP, _H), jnp.float32)] * 2,
    )(x, W1_rel, W1_root)

    parts1 = _sc_segsum_h(xw1, ei, edge_attr)

    hw2, hr2 = pl.pallas_call(
        _dense2_body,
        grid=(grid,),
        in_specs=[
            pl.BlockSpec((_NC, _BN, _H), lambda i: (0, i, 0)),
            pl.BlockSpec((_BN, _H), lambda i: (i, 0)),
            pl.BlockSpec((1, _H), lambda i: (0, 0)),
            pl.BlockSpec((_H, _C), lambda i: (0, 0)),
            pl.BlockSpec((_H, _C), lambda i: (0, 0)),
        ],
        out_specs=[
            pl.BlockSpec((_BN, _C), lambda i: (i, 0)),
            pl.BlockSpec((_BN, _C), lambda i: (i, 0)),
        ],
        out_shape=[jax.ShapeDtypeStruct((_NP, _C), jnp.float32)] * 2,
    )(parts1, xr1, b1.reshape(1, _H), W2_rel, W2_root)

    parts2 = _sc_segsum_c(hw2, ei, edge_attr)

    out = pl.pallas_call(
        _out_body,
        grid=(grid,),
        in_specs=[
            pl.BlockSpec((_NC, _BN, _C), lambda i: (0, i, 0)),
            pl.BlockSpec((_BN, _C), lambda i: (i, 0)),
            pl.BlockSpec((1, _C), lambda i: (0, 0)),
        ],
        out_specs=pl.BlockSpec((_BN, _C), lambda i: (i, 0)),
        out_shape=jax.ShapeDtypeStruct((_N, _C), jnp.float32),
    )(parts2, hr2, b2.reshape(1, _C))

    return out


# speedup vs baseline: 1.0498x; 1.0498x over previous
import functools

import jax
import jax.numpy as jnp
from jax import lax
from jax.experimental import pallas as pl
from jax.experimental.pallas import tpu as pltpu
from jax.experimental.pallas import tpu_sc as plsc

_N = 10000
_E = 320000
_F_IN = 128
_H = 32
_C = 16

_NC = 2
_NS = 16
_NW = _NC * _NS
_EPW = _E // _NW
_CH = 128
_NCHUNK = _EPW // _CH
_TAIL = _EPW - _NCHUNK * _CH
_NBUF = 6
_DEPTH = 3
_NP = 10240
_RPS = _NP // _NS


def _make_sc_segsum(K: int):
    mesh = plsc.VectorSubcoreMesh(
        core_axis_name="c", subcore_axis_name="s",
        num_cores=_NC, num_subcores=_NS)

    def body(t_hbm, ei_hbm, ea_hbm, out_hbm,
             src_v, dst_v, ea_v, rows, acc, t_sp, gsem, ssem):
        cid = lax.axis_index("c")
        sid = lax.axis_index("s")
        wid = sid * _NC + cid

        def zrow(r, _):
            for f in range(0, K, 16):
                rows[0][r, pl.ds(f, 16)] = jnp.zeros((16,), jnp.float32)
            return 0

        lax.fori_loop(0, _CH, zrow, 0)
        for i in range(_RPS // _CH):
            pltpu.sync_copy(rows[0],
                            acc.at[pl.ds(sid * _RPS + i * _CH, _CH)])

        pltpu.sync_copy(t_hbm.at[pl.ds(sid * _RPS, _RPS)],
                        t_sp.at[pl.ds(sid * _RPS, _RPS)])

        base = wid * _EPW
        pltpu.sync_copy(ei_hbm.at[0, pl.ds(base, _EPW)], src_v)
        pltpu.sync_copy(ei_hbm.at[1, pl.ds(base, _EPW)], dst_v)
        pltpu.sync_copy(ea_hbm.at[pl.ds(base, _EPW)], ea_v)
        plsc.subcore_barrier()

        for b in range(_DEPTH):
            pltpu.async_copy(t_sp.at[src_v.at[pl.ds(b * _CH, _CH)]], rows[b], gsem[b])

        def step(c, b):
            pltpu.make_async_copy(
                t_sp.at[src_v.at[pl.ds(c * _CH, _CH)]], rows[b], gsem[b]).wait()
            for g in range(0, _CH, 16):
                ev = ea_v[pl.ds(c * _CH + g, 16)]
                for j in range(16):
                    r = g + j
                    s = ev[j]
                    for f in range(0, K, 16):
                        rows[b][r, pl.ds(f, 16)] = rows[b][r, pl.ds(f, 16)] * s
            pltpu.async_copy(rows[b], acc.at[dst_v.at[pl.ds(c * _CH, _CH)]], ssem[b],
                             add=True)
            cn = c + _DEPTH
            bn = (b + _DEPTH) % _NBUF
            co = c - (_NBUF - _DEPTH)

            @pl.when(cn < _NCHUNK)
            def _():
                @pl.when(co >= 0)
                def _():
                    pltpu.make_async_copy(
                        rows[bn], acc.at[dst_v.at[pl.ds(co * _CH, _CH)]], ssem[bn]).wait()
                pltpu.async_copy(t_sp.at[src_v.at[pl.ds(cn * _CH, _CH)]], rows[bn], gsem[bn])

        def macro(m, _):
            for b in range(_NBUF):
                step(m * _NBUF + b, b)
            return 0

        lax.fori_loop(0, _NCHUNK // _NBUF, macro, 0)

        for b in range(_NBUF):
            cc = _NCHUNK - _NBUF + b
            pltpu.make_async_copy(
                rows[b], acc.at[dst_v.at[pl.ds(cc * _CH, _CH)]], ssem[b]).wait()

        tb = _NCHUNK * _CH
        pltpu.async_copy(
            t_sp.at[src_v.at[pl.ds(tb, _TAIL)]],
            rows[0].at[pl.ds(0, _TAIL)], gsem[0]).wait()
        ev = ea_v[pl.ds(tb, 16)]
        for j in range(_TAIL):
            for f in range(0, K, 16):
                rows[0][j, pl.ds(f, 16)] = rows[0][j, pl.ds(f, 16)] * ev[j]
        pltpu.async_copy(
            rows[0].at[pl.ds(0, _TAIL)],
            acc.at[dst_v.at[pl.ds(tb, _TAIL)]], ssem[0], add=True).wait()

        plsc.subcore_barrier()

        pltpu.sync_copy(acc.at[pl.ds(sid * _RPS, _RPS)],
                        out_hbm.at[cid, pl.ds(sid * _RPS, _RPS)])

    return pl.kernel(
        body,
        out_type=jax.ShapeDtypeStruct((_NC, _NP, K), jnp.float32),
        mesh=mesh,
        compiler_params=pltpu.CompilerParams(use_tc_tiling_on_sc=False),
        scratch_types=[
            pltpu.VMEM((_EPW,), jnp.int32),
            pltpu.VMEM((_EPW,), jnp.int32),
            pltpu.VMEM((_EPW,), jnp.float32),
            [pltpu.VMEM((_CH, K), jnp.float32) for _ in range(_NBUF)],
            pltpu.VMEM_SHARED((_NP, K), jnp.float32),
            pltpu.VMEM_SHARED((_NP, K), jnp.float32),
            [pltpu.SemaphoreType.DMA for _ in range(_NBUF)],
            [pltpu.SemaphoreType.DMA for _ in range(_NBUF)],
        ],
        name=f"sc_segsum_k{K}",
    )


_sc_segsum_h = _make_sc_segsum(_H)
_sc_segsum_c = _make_sc_segsum(_C)


def _dense1_body(x_ref, wrel_ref, wroot_ref, xw_ref, xr_ref):
    x = x_ref[...]
    xw_ref[...] = jnp.dot(x, wrel_ref[...],
                          preferred_element_type=jnp.float32)
    xr_ref[...] = jnp.dot(x, wroot_ref[...],
                          preferred_element_type=jnp.float32)


def _dense2_body(parts_ref, xr_ref, b1_ref, wrel_ref, wroot_ref,
                 hw_ref, hr_ref):
    s = parts_ref[0] + parts_ref[1] + xr_ref[...] + b1_ref[...]
    h = jnp.maximum(s, 0.0)
    hw_ref[...] = jnp.dot(h, wrel_ref[...],
                          preferred_element_type=jnp.float32)
    hr_ref[...] = jnp.dot(h, wroot_ref[...],
                          preferred_element_type=jnp.float32)


def _out_body(parts_ref, hr_ref, b2_ref, o_ref):
    t = parts_ref[0] + parts_ref[1] + hr_ref[...] + b2_ref[...]
    m = jnp.max(t, axis=1, keepdims=True)
    lse = jnp.log(jnp.sum(jnp.exp(t - m), axis=1, keepdims=True)) + m
    o_ref[...] = t - lse


_BN = 2000


def kernel(x, edge_index, edge_attr, W1_rel, b1, W1_root, W2_rel, b2,
           W2_root):
    ei = edge_index.astype(jnp.int32)
    grid = _N // _BN

    xw1, xr1 = pl.pallas_call(
        _dense1_body,
        grid=(grid,),
        in_specs=[
            pl.BlockSpec((_BN, _F_IN), lambda i: (i, 0)),
            pl.BlockSpec((_F_IN, _H), lambda i: (0, 0)),
            pl.BlockSpec((_F_IN, _H), lambda i: (0, 0)),
        ],
        out_specs=[
            pl.BlockSpec((_BN, _H), lambda i: (i, 0)),
            pl.BlockSpec((_BN, _H), lambda i: (i, 0)),
        ],
        out_shape=[jax.ShapeDtypeStruct((_NP, _H), jnp.float32)] * 2,
    )(x, W1_rel, W1_root)

    parts1 = _sc_segsum_h(xw1, ei, edge_attr)

    hw2, hr2 = pl.pallas_call(
        _dense2_body,
        grid=(grid,),
        in_specs=[
            pl.BlockSpec((_NC, _BN, _H), lambda i: (0, i, 0)),
            pl.BlockSpec((_BN, _H), lambda i: (i, 0)),
            pl.BlockSpec((1, _H), lambda i: (0, 0)),
            pl.BlockSpec((_H, _C), lambda i: (0, 0)),
            pl.BlockSpec((_H, _C), lambda i: (0, 0)),
        ],
        out_specs=[
            pl.BlockSpec((_BN, _C), lambda i: (i, 0)),
            pl.BlockSpec((_BN, _C), lambda i: (i, 0)),
        ],
        out_shape=[jax.ShapeDtypeStruct((_NP, _C), jnp.float32)] * 2,
    )(parts1, xr1, b1.reshape(1, _H), W2_rel, W2_root)

    parts2 = _sc_segsum_c(hw2, ei, edge_attr)

    out = pl.pallas_call(
        _out_body,
        grid=(grid,),
        in_specs=[
            pl.BlockSpec((_NC, _BN, _C), lambda i: (0, i, 0)),
            pl.BlockSpec((_BN, _C), lambda i: (i, 0)),
            pl.BlockSpec((1, _C), lambda i: (0, 0)),
        ],
        out_specs=pl.BlockSpec((_BN, _C), lambda i: (i, 0)),
        out_shape=jax.ShapeDtypeStruct((_N, _C), jnp.float32),
    )(parts2, hr2, b2.reshape(1, _C))

    return out


# speedup vs baseline: 1.1473x; 1.0929x over previous
import functools

import jax
import jax.numpy as jnp
from jax import lax
from jax.experimental import pallas as pl
from jax.experimental.pallas import tpu as pltpu
from jax.experimental.pallas import tpu_sc as plsc

_N = 10000
_E = 320000
_F_IN = 128
_H = 32
_C = 16

_NC = 2
_NS = 16
_NW = _NC * _NS
_EPW = _E // _NW
_CH = 80
_NCHUNK = _EPW // _CH
_NBUF = 5
_DEPTH = 3
_NP = 10240
_RPS = _NP // _NS


def _make_sc_segsum(K: int):
    mesh = plsc.VectorSubcoreMesh(
        core_axis_name="c", subcore_axis_name="s",
        num_cores=_NC, num_subcores=_NS)

    def body(t_hbm, ei_hbm, ea_hbm, out_hbm,
             src_v, dst_v, ea_v, rows, acc, t_sp, gsem, ssem):
        cid = lax.axis_index("c")
        sid = lax.axis_index("s")
        wid = sid * _NC + cid

        def zrow(r, _):
            for f in range(0, K, 16):
                rows[0][r, pl.ds(f, 16)] = jnp.zeros((16,), jnp.float32)
            return 0

        lax.fori_loop(0, _CH, zrow, 0)
        for i in range(_RPS // _CH):
            pltpu.sync_copy(rows[0],
                            acc.at[pl.ds(sid * _RPS + i * _CH, _CH)])

        pltpu.sync_copy(t_hbm.at[pl.ds(sid * _RPS, _RPS)],
                        t_sp.at[pl.ds(sid * _RPS, _RPS)])

        base = wid * _EPW
        pltpu.sync_copy(ei_hbm.at[0, pl.ds(base, _EPW)], src_v)
        pltpu.sync_copy(ei_hbm.at[1, pl.ds(base, _EPW)], dst_v)
        pltpu.sync_copy(ea_hbm.at[pl.ds(base, _EPW)], ea_v)
        plsc.subcore_barrier()

        for b in range(_DEPTH):
            pltpu.async_copy(t_sp.at[src_v.at[pl.ds(b * _CH, _CH)]], rows[b], gsem[b])

        def step(c, b):
            pltpu.make_async_copy(
                t_sp.at[src_v.at[pl.ds(c * _CH, _CH)]], rows[b], gsem[b]).wait()
            for g in range(0, _CH, 16):
                ev = ea_v[pl.ds(c * _CH + g, 16)]
                for j in range(16):
                    r = g + j
                    s = ev[j]
                    for f in range(0, K, 16):
                        rows[b][r, pl.ds(f, 16)] = rows[b][r, pl.ds(f, 16)] * s
            pltpu.async_copy(rows[b], acc.at[dst_v.at[pl.ds(c * _CH, _CH)]], ssem[b],
                             add=True)
            cn = c + _DEPTH
            bn = (b + _DEPTH) % _NBUF
            co = c - (_NBUF - _DEPTH)

            @pl.when(cn < _NCHUNK)
            def _():
                @pl.when(co >= 0)
                def _():
                    pltpu.make_async_copy(
                        rows[bn], acc.at[dst_v.at[pl.ds(co * _CH, _CH)]], ssem[bn]).wait()
                pltpu.async_copy(t_sp.at[src_v.at[pl.ds(cn * _CH, _CH)]], rows[bn], gsem[bn])

        def macro(m, _):
            for b in range(_NBUF):
                step(m * _NBUF + b, b)
            return 0

        lax.fori_loop(0, _NCHUNK // _NBUF, macro, 0)

        for b in range(_NBUF):
            cc = _NCHUNK - _NBUF + b
            pltpu.make_async_copy(
                rows[b], acc.at[dst_v.at[pl.ds(cc * _CH, _CH)]], ssem[b]).wait()

        plsc.subcore_barrier()

        pltpu.sync_copy(acc.at[pl.ds(sid * _RPS, _RPS)],
                        out_hbm.at[cid, pl.ds(sid * _RPS, _RPS)])

    return pl.kernel(
        body,
        out_type=jax.ShapeDtypeStruct((_NC, _NP, K), jnp.float32),
        mesh=mesh,
        compiler_params=pltpu.CompilerParams(use_tc_tiling_on_sc=False),
        scratch_types=[
            pltpu.VMEM((_EPW,), jnp.int32),
            pltpu.VMEM((_EPW,), jnp.int32),
            pltpu.VMEM((_EPW,), jnp.float32),
            [pltpu.VMEM((_CH, K), jnp.float32) for _ in range(_NBUF)],
            pltpu.VMEM_SHARED((_NP, K), jnp.float32),
            pltpu.VMEM_SHARED((_NP, K), jnp.float32),
            [pltpu.SemaphoreType.DMA for _ in range(_NBUF)],
            [pltpu.SemaphoreType.DMA for _ in range(_NBUF)],
        ],
        name=f"sc_segsum_k{K}",
    )


_sc_segsum_h = _make_sc_segsum(_H)
_sc_segsum_c = _make_sc_segsum(_C)


def _dense1_body(x_ref, wrel_ref, wroot_ref, xw_ref, xr_ref):
    x = x_ref[...]
    xw_ref[...] = jnp.dot(x, wrel_ref[...],
                          preferred_element_type=jnp.float32)
    xr_ref[...] = jnp.dot(x, wroot_ref[...],
                          preferred_element_type=jnp.float32)


def _dense2_body(parts_ref, xr_ref, b1_ref, wrel_ref, wroot_ref,
                 hw_ref, hr_ref):
    s = parts_ref[0] + parts_ref[1] + xr_ref[...] + b1_ref[...]
    h = jnp.maximum(s, 0.0)
    hw_ref[...] = jnp.dot(h, wrel_ref[...],
                          preferred_element_type=jnp.float32)
    hr_ref[...] = jnp.dot(h, wroot_ref[...],
                          preferred_element_type=jnp.float32)


def _out_body(parts_ref, hr_ref, b2_ref, o_ref):
    t = parts_ref[0] + parts_ref[1] + hr_ref[...] + b2_ref[...]
    m = jnp.max(t, axis=1, keepdims=True)
    lse = jnp.log(jnp.sum(jnp.exp(t - m), axis=1, keepdims=True)) + m
    o_ref[...] = t - lse


_BN = 2000


def kernel(x, edge_index, edge_attr, W1_rel, b1, W1_root, W2_rel, b2,
           W2_root):
    ei = edge_index.astype(jnp.int32)
    grid = _N // _BN

    xw1, xr1 = pl.pallas_call(
        _dense1_body,
        grid=(grid,),
        in_specs=[
            pl.BlockSpec((_BN, _F_IN), lambda i: (i, 0)),
            pl.BlockSpec((_F_IN, _H), lambda i: (0, 0)),
            pl.BlockSpec((_F_IN, _H), lambda i: (0, 0)),
        ],
        out_specs=[
            pl.BlockSpec((_BN, _H), lambda i: (i, 0)),
            pl.BlockSpec((_BN, _H), lambda i: (i, 0)),
        ],
        out_shape=[jax.ShapeDtypeStruct((_NP, _H), jnp.float32)] * 2,
    )(x, W1_rel, W1_root)

    parts1 = _sc_segsum_h(xw1, ei, edge_attr)

    hw2, hr2 = pl.pallas_call(
        _dense2_body,
        grid=(grid,),
        in_specs=[
            pl.BlockSpec((_NC, _BN, _H), lambda i: (0, i, 0)),
            pl.BlockSpec((_BN, _H), lambda i: (i, 0)),
            pl.BlockSpec((1, _H), lambda i: (0, 0)),
            pl.BlockSpec((_H, _C), lambda i: (0, 0)),
            pl.BlockSpec((_H, _C), lambda i: (0, 0)),
        ],
        out_specs=[
            pl.BlockSpec((_BN, _C), lambda i: (i, 0)),
            pl.BlockSpec((_BN, _C), lambda i: (i, 0)),
        ],
        out_shape=[jax.ShapeDtypeStruct((_NP, _C), jnp.float32)] * 2,
    )(parts1, xr1, b1.reshape(1, _H), W2_rel, W2_root)

    parts2 = _sc_segsum_c(hw2, ei, edge_attr)

    out = pl.pallas_call(
        _out_body,
        grid=(grid,),
        in_specs=[
            pl.BlockSpec((_NC, _BN, _C), lambda i: (0, i, 0)),
            pl.BlockSpec((_BN, _C), lambda i: (i, 0)),
            pl.BlockSpec((1, _C), lambda i: (0, 0)),
        ],
        out_specs=pl.BlockSpec((_BN, _C), lambda i: (i, 0)),
        out_shape=jax.ShapeDtypeStruct((_N, _C), jnp.float32),
    )(parts2, hr2, b2.reshape(1, _C))

    return out
